# baseline (speedup 1.0000x reference)
import jax
import jax.numpy as jnp
from jax import lax
from jax.experimental import pallas as pl
from jax.experimental.pallas import tpu as pltpu
from jax.experimental.pallas import tpu_sc as plsc

NUM_ACTIONS = 1000
EMBED_DIM = 64
PAD_DIM = 128
BATCH = 16384

NUM_CORES = 2
NUM_SUBCORES = 16
NUM_WORKERS = NUM_CORES * NUM_SUBCORES
B_PER_W = BATCH // NUM_WORKERS
N_CHUNKS = 4


def _gather_body(idx_hbm, table_hbm, out_hbm, idx_v, rows_v, gsems, sem):
    sid = lax.axis_index("s")
    wid = sid * NUM_CORES + lax.axis_index("c")
    base = wid * B_PER_W
    idx_copy = pltpu.async_copy(idx_hbm.at[pl.ds(base, B_PER_W)], idx_v, sem)
    idx_copy.wait()
    chunk = B_PER_W // N_CHUNKS
    gathers = [
        pltpu.async_copy(
            table_hbm.at[idx_v.at[pl.ds(j * chunk, chunk)]],
            rows_v.at[pl.ds(j * chunk, chunk)],
            gsems.at[j],
        )
        for j in range(N_CHUNKS)
    ]
    writes = []
    for j in range(N_CHUNKS):
        gathers[j].wait()
        writes.append(
            pltpu.async_copy(
                rows_v.at[pl.ds(j * chunk, chunk)],
                out_hbm.at[pl.ds(base + j * chunk, chunk)],
                sem,
            )
        )
    for w in writes:
        w.wait()


@jax.jit
def _lookup(action_ids, embed_table):
    mesh = plsc.VectorSubcoreMesh(core_axis_name="c", subcore_axis_name="s")
    run = pl.kernel(
        _gather_body,
        out_type=jax.ShapeDtypeStruct((BATCH, PAD_DIM), jnp.float32),
        mesh=mesh,
        scratch_types=[
            pltpu.VMEM((B_PER_W,), jnp.int32),
            pltpu.VMEM((B_PER_W, PAD_DIM), jnp.float32),
            pltpu.SemaphoreType.DMA((N_CHUNKS,)),
            pltpu.SemaphoreType.DMA,
        ],
    )
    table_padded = jnp.pad(embed_table, ((0, 0), (0, PAD_DIM - EMBED_DIM)))
    return run(action_ids, table_padded)[:, :EMBED_DIM]


def kernel(action_ids, embed_table):
    return _lookup(action_ids.astype(jnp.int32), embed_table)


# speedup vs baseline: 1.1490x; 1.1490x over previous
import jax
import jax.numpy as jnp
from jax import lax
from jax.experimental import pallas as pl
from jax.experimental.pallas import tpu as pltpu
from jax.experimental.pallas import tpu_sc as plsc

NUM_ACTIONS = 1000
EMBED_DIM = 64
PAD_DIM = 128
BATCH = 16384

NUM_CORES = 2
NUM_SUBCORES = 16
NUM_WORKERS = NUM_CORES * NUM_SUBCORES
B_PER_W = BATCH // NUM_WORKERS
N_CHUNKS = 4


def _gather_body(idx_hbm, table_hbm, out_hbm, table_sh, idx_v, rows_v, gsems, sem):
    sid = lax.axis_index("s")
    wid = sid * NUM_CORES + lax.axis_index("c")
    base = wid * B_PER_W
    @pl.when(sid < 7)
    def _():
        pltpu.sync_copy(
            table_hbm.at[pl.ds(sid * 128, 128)], table_sh.at[pl.ds(sid * 128, 128)]
        )

    @pl.when(sid == 7)
    def _():
        pltpu.sync_copy(
            table_hbm.at[pl.ds(896, NUM_ACTIONS - 896)],
            table_sh.at[pl.ds(896, NUM_ACTIONS - 896)],
        )

    idx_copy = pltpu.async_copy(idx_hbm.at[pl.ds(base, B_PER_W)], idx_v, sem)
    plsc.subcore_barrier()
    idx_copy.wait()
    chunk = B_PER_W // N_CHUNKS
    gathers = [
        pltpu.async_copy(
            table_sh.at[idx_v.at[pl.ds(j * chunk, chunk)]],
            rows_v.at[pl.ds(j * chunk, chunk)],
            gsems.at[j],
        )
        for j in range(N_CHUNKS)
    ]
    writes = []
    for j in range(N_CHUNKS):
        gathers[j].wait()
        writes.append(
            pltpu.async_copy(
                rows_v.at[pl.ds(j * chunk, chunk)],
                out_hbm.at[pl.ds(base + j * chunk, chunk)],
                sem,
            )
        )
    for w in writes:
        w.wait()


@jax.jit
def _lookup(action_ids, embed_table):
    mesh = plsc.VectorSubcoreMesh(core_axis_name="c", subcore_axis_name="s")
    run = pl.kernel(
        _gather_body,
        out_type=jax.ShapeDtypeStruct((BATCH, PAD_DIM), jnp.float32),
        mesh=mesh,
        scratch_types=[
            pltpu.VMEM_SHARED((NUM_ACTIONS, PAD_DIM), jnp.float32),
            pltpu.VMEM((B_PER_W,), jnp.int32),
            pltpu.VMEM((B_PER_W, PAD_DIM), jnp.float32),
            pltpu.SemaphoreType.DMA((N_CHUNKS,)),
            pltpu.SemaphoreType.DMA,
        ],
    )
    table_padded = jnp.pad(embed_table, ((0, 0), (0, PAD_DIM - EMBED_DIM)))
    return run(action_ids, table_padded)[:, :EMBED_DIM]


def kernel(action_ids, embed_table):
    return _lookup(action_ids.astype(jnp.int32), embed_table)


# speedup vs baseline: 1.1511x; 1.0019x over previous
import jax
import jax.numpy as jnp
from jax import lax
from jax.experimental import pallas as pl
from jax.experimental.pallas import tpu as pltpu
from jax.experimental.pallas import tpu_sc as plsc

NUM_ACTIONS = 1000
EMBED_DIM = 64
PAD_DIM = 128
BATCH = 16384

NUM_CORES = 2
NUM_SUBCORES = 16
NUM_WORKERS = NUM_CORES * NUM_SUBCORES
B_PER_W = BATCH // NUM_WORKERS
N_CHUNKS = 4


def _gather_body(idx_hbm, table_hbm, out_hbm, table_sh, idx_v, rows_v, gsems, sem):
    sid = lax.axis_index("s")
    wid = sid * NUM_CORES + lax.axis_index("c")
    base = wid * B_PER_W
    @pl.when(sid == 0)
    def _():
        pltpu.sync_copy(table_hbm, table_sh)

    idx_copy = pltpu.async_copy(idx_hbm.at[pl.ds(base, B_PER_W)], idx_v, sem)
    plsc.subcore_barrier()
    idx_copy.wait()
    chunk = B_PER_W // N_CHUNKS
    gathers = [
        pltpu.async_copy(
            table_sh.at[idx_v.at[pl.ds(j * chunk, chunk)]],
            rows_v.at[pl.ds(j * chunk, chunk)],
            gsems.at[j],
        )
        for j in range(N_CHUNKS)
    ]
    writes = []
    for j in range(N_CHUNKS):
        gathers[j].wait()
        writes.append(
            pltpu.async_copy(
                rows_v.at[pl.ds(j * chunk, chunk)],
                out_hbm.at[pl.ds(base + j * chunk, chunk)],
                sem,
            )
        )
    for w in writes:
        w.wait()


@jax.jit
def _lookup(action_ids, embed_table):
    mesh = plsc.VectorSubcoreMesh(core_axis_name="c", subcore_axis_name="s")
    run = pl.kernel(
        _gather_body,
        out_type=jax.ShapeDtypeStruct((BATCH, PAD_DIM), jnp.float32),
        mesh=mesh,
        scratch_types=[
            pltpu.VMEM_SHARED((NUM_ACTIONS, PAD_DIM), jnp.float32),
            pltpu.VMEM((B_PER_W,), jnp.int32),
            pltpu.VMEM((B_PER_W, PAD_DIM), jnp.float32),
            pltpu.SemaphoreType.DMA((N_CHUNKS,)),
            pltpu.SemaphoreType.DMA,
        ],
    )
    table_padded = jnp.pad(embed_table, ((0, 0), (0, PAD_DIM - EMBED_DIM)))
    return run(action_ids, table_padded)[:, :EMBED_DIM]


def kernel(action_ids, embed_table):
    return _lookup(action_ids.astype(jnp.int32), embed_table)
